# baseline (speedup 1.0000x reference)
import functools

import jax
import jax.numpy as jnp
from jax import lax
from jax.experimental import pallas as pl
from jax.experimental.pallas import tpu as pltpu
from jax.experimental.pallas import tpu_sc as plsc

N = 10000
E = 320000
D = 128
DOUT = 40
NC, NS, LANES = 2, 16, 16
HALF = 5120
NJ = 128
NACC = HALF + NJ
EPT = E // NS
K = 80
NCH = EPT // K
RPA = NACC // NS
RPC = HALF // NS
NPAD = NC * HALF

_MESH = plsc.VectorSubcoreMesh(core_axis_name="c", subcore_axis_name="s")

_SC_SCRATCH = [
    pltpu.VMEM((NCH, K), jnp.int32),
    pltpu.VMEM((NCH, K), jnp.int32),
    pltpu.VMEM((K, D), jnp.float32),
    pltpu.VMEM((K, D), jnp.float32),
    pltpu.VMEM((K, D), jnp.float32),
    pltpu.VMEM_SHARED((NACC, D), jnp.float32),
    pltpu.SemaphoreType.DMA,
    pltpu.SemaphoreType.DMA,
]


def _fill_rows(buf, value):
    @pl.loop(0, K)
    def _(r):
        for c in range(0, D, LANES):
            buf[r, pl.ds(c, LANES)] = jnp.full((LANES,), value, jnp.float32)


def _zero_acc(zbuf, acc_sh, sid):
    rbase = sid * RPA
    @pl.loop(0, RPA // K)
    def _(j):
        pltpu.sync_copy(zbuf, acc_sh.at[pl.ds(rbase + j * K, K)])
    pltpu.sync_copy(zbuf.at[pl.ds(0, RPA % K)],
                    acc_sh.at[pl.ds(rbase + (RPA // K) * K, RPA % K)])


def _copyout(acc_sh, out_hbm, cid, sid):
    pltpu.sync_copy(acc_sh.at[pl.ds(sid * RPC, RPC)],
                    out_hbm.at[pl.ds(cid * HALF + sid * RPC, RPC)])


@functools.partial(
    pl.kernel,
    out_type=jax.ShapeDtypeStruct((NPAD, D), jnp.float32),
    mesh=_MESH,
    scratch_types=_SC_SCRATCH,
)
def _sc_agg(g_hbm, row_hbm, col_hbm, out_hbm, ridx, cidx, buf0, buf1,
            zbuf, acc_sh, sem0, sem1):
    cid = lax.axis_index("c")
    sid = lax.axis_index("s")

    def gather(c, buf, sem):
        return pltpu.make_async_copy(g_hbm.at[ridx.at[c]], buf, sem)

    pltpu.sync_copy(row_hbm.at[cid].at[sid], ridx)
    pltpu.sync_copy(col_hbm.at[cid].at[sid], cidx)
    _fill_rows(zbuf, 0.0)
    _zero_acc(zbuf, acc_sh, sid)
    plsc.subcore_barrier()

    @pl.loop(0, NCH)
    def _(i):
        pltpu.async_copy(g_hbm.at[ridx.at[i]], buf0, sem0).wait()
        pltpu.sync_copy(buf0, acc_sh.at[cidx.at[i]], add=True)

    plsc.subcore_barrier()
    _copyout(acc_sh, out_hbm, cid, sid)


@functools.partial(
    pl.kernel,
    out_type=jax.ShapeDtypeStruct((NPAD, D), jnp.float32),
    mesh=_MESH,
    scratch_types=_SC_SCRATCH,
)
def _sc_deg(row_hbm, col_hbm, out_hbm, ridx, cidx, buf, buf1, zbuf,
            acc_sh, sem0, sem1):
    cid = lax.axis_index("c")
    sid = lax.axis_index("s")

    pltpu.sync_copy(col_hbm.at[cid].at[sid], cidx)
    _fill_rows(buf, 1.0)
    _fill_rows(zbuf, 0.0)
    _zero_acc(zbuf, acc_sh, sid)
    plsc.subcore_barrier()

    @pl.loop(0, NCH)
    def _(i):
        pltpu.sync_copy(buf, acc_sh.at[cidx.at[i]], add=True)

    plsc.subcore_barrier()
    _copyout(acc_sh, out_hbm, cid, sid)


BM = 1000
EROWS = E // D


def _clamp_body(row_ref, col_ref, ro_ref, co_ref):
    row = row_ref[...]
    col = col_ref[...]
    junk = HALF + (col & (NJ - 1))
    for c in range(NC):
        lcl = col - c * HALF
        ok = (lcl >= 0) & (lcl < HALF)
        ro_ref[c] = jnp.where(ok, row, 0)
        co_ref[c] = jnp.where(ok, lcl, junk)


def _tc_clamp(row2d, col2d):
    return pl.pallas_call(
        _clamp_body,
        grid=(1,),
        in_specs=[pl.BlockSpec((EROWS, D), lambda i: (0, 0)),
                  pl.BlockSpec((EROWS, D), lambda i: (0, 0))],
        out_specs=[pl.BlockSpec((NC, EROWS, D), lambda i: (0, 0, 0)),
                   pl.BlockSpec((NC, EROWS, D), lambda i: (0, 0, 0))],
        out_shape=[jax.ShapeDtypeStruct((NC, EROWS, D), jnp.int32),
                   jax.ShapeDtypeStruct((NC, EROWS, D), jnp.int32)],
    )(row2d, col2d)


def _mm_body(x_ref, w_ref, o_ref):
    o_ref[...] = jnp.dot(x_ref[...], w_ref[...],
                         preferred_element_type=jnp.float32)


def _tc_mm(x, W):
    return pl.pallas_call(
        _mm_body,
        grid=(N // BM,),
        in_specs=[pl.BlockSpec((BM, D), lambda i: (i, 0)),
                  pl.BlockSpec((D, D), lambda i: (0, 0))],
        out_specs=pl.BlockSpec((BM, D), lambda i: (i, 0)),
        out_shape=jax.ShapeDtypeStruct((N, D), jnp.float32),
    )(x, W)


def _prep_body(degc_ref, h_ref, dinv_ref, g_ref):
    deg = degc_ref[:, 0:1] + 1.0
    dinv = lax.rsqrt(deg)
    dinv_ref[...] = dinv
    g_ref[...] = dinv * h_ref[...]


def _tc_prep(degc, H1):
    return pl.pallas_call(
        _prep_body,
        grid=(N // BM,),
        in_specs=[pl.BlockSpec((BM, D), lambda i: (i, 0)),
                  pl.BlockSpec((BM, D), lambda i: (i, 0))],
        out_specs=[pl.BlockSpec((BM, 1), lambda i: (i, 0)),
                   pl.BlockSpec((BM, D), lambda i: (i, 0))],
        out_shape=[jax.ShapeDtypeStruct((N, 1), jnp.float32),
                   jax.ShapeDtypeStruct((N, D), jnp.float32)],
    )(degc, H1)


def _mid_mm_body(s_ref, g_ref, dinv_ref, b_ref, w_ref, o_ref):
    dinv = dinv_ref[...]
    t = jnp.maximum(dinv * (s_ref[...] + g_ref[...]) + b_ref[...], 0.0)
    o_ref[...] = dinv * jnp.dot(t, w_ref[...],
                                preferred_element_type=jnp.float32)


def _mid_nomm_body(s_ref, g_ref, dinv_ref, b_ref, o_ref):
    dinv = dinv_ref[...]
    t = jnp.maximum(dinv * (s_ref[...] + g_ref[...]) + b_ref[...], 0.0)
    o_ref[...] = dinv * t


def _tc_mid(S, g, dinv, b, W=None):
    specs = [pl.BlockSpec((BM, D), lambda i: (i, 0)),
             pl.BlockSpec((BM, D), lambda i: (i, 0)),
             pl.BlockSpec((BM, 1), lambda i: (i, 0)),
             pl.BlockSpec((1, D), lambda i: (0, 0))]
    args = [S, g, dinv, b.reshape(1, D)]
    if W is not None:
        specs.append(pl.BlockSpec((D, D), lambda i: (0, 0)))
        args.append(W)
    return pl.pallas_call(
        _mid_mm_body if W is not None else _mid_nomm_body,
        grid=(N // BM,),
        in_specs=specs,
        out_specs=pl.BlockSpec((BM, D), lambda i: (i, 0)),
        out_shape=jax.ShapeDtypeStruct((N, D), jnp.float32),
    )(*args)


def _fin_body(s_ref, g_ref, dinv_ref, w_ref, b_ref, o_ref):
    pre = dinv_ref[...] * (s_ref[...] + g_ref[...])
    o_ref[...] = (jnp.dot(pre, w_ref[...], preferred_element_type=jnp.float32)
                  + b_ref[...])


def _tc_fin(S3, g3, dinv, W3, b3):
    return pl.pallas_call(
        _fin_body,
        grid=(N // BM,),
        in_specs=[pl.BlockSpec((BM, D), lambda i: (i, 0)),
                  pl.BlockSpec((BM, D), lambda i: (i, 0)),
                  pl.BlockSpec((BM, 1), lambda i: (i, 0)),
                  pl.BlockSpec((D, DOUT), lambda i: (0, 0)),
                  pl.BlockSpec((1, DOUT), lambda i: (0, 0))],
        out_specs=pl.BlockSpec((BM, DOUT), lambda i: (i, 0)),
        out_shape=jax.ShapeDtypeStruct((N, DOUT), jnp.float32),
    )(S3, g3, dinv, W3, b3.reshape(1, DOUT))


def kernel(x, edge_index, W1, b1, W2, b2, W3, b3):
    row = edge_index[0].astype(jnp.int32)
    col = edge_index[1].astype(jnp.int32)
    rowc, colc = _tc_clamp(row.reshape(EROWS, D), col.reshape(EROWS, D))
    rowt = rowc.reshape(NC, NS, NCH, K)
    colc = colc.reshape(NC, NS, NCH, K)

    degf = _sc_deg(rowt, colc)
    H1 = _tc_mm(x, W1)
    dinv, g1 = _tc_prep(degf[:N], H1)
    S1 = _sc_agg(g1, rowt, colc)
    g2 = _tc_mid(S1[:N], g1, dinv, b1, W2)
    S2 = _sc_agg(g2, rowt, colc)
    g3 = _tc_mid(S2[:N], g2, dinv, b2)
    S3 = _sc_agg(g3, rowt, colc)
    return _tc_fin(S3[:N], g3, dinv, W3, b3)


# speedup vs baseline: 26.4324x; 26.4324x over previous
import functools

import jax
import jax.numpy as jnp
from jax import lax
from jax.experimental import pallas as pl
from jax.experimental.pallas import tpu as pltpu
from jax.experimental.pallas import tpu_sc as plsc

N = 10000
E = 320000
D = 128
DOUT = 40
NC, NS, LANES = 2, 16, 16
HALF = 5120
NJ = 128
NACC = HALF + NJ
EPT = E // NS
K = 80
NCH = EPT // K
RPA = NACC // NS
RPC = HALF // NS
NPAD = NC * HALF

_MESH = plsc.VectorSubcoreMesh(core_axis_name="c", subcore_axis_name="s")

_SC_SCRATCH = [
    pltpu.VMEM((NCH, K), jnp.int32),
    pltpu.VMEM((NCH, K), jnp.int32),
    pltpu.VMEM((K, D), jnp.float32),
    pltpu.VMEM((K, D), jnp.float32),
    pltpu.VMEM((K, D), jnp.float32),
    pltpu.VMEM_SHARED((NACC, D), jnp.float32),
    pltpu.SemaphoreType.DMA,
    pltpu.SemaphoreType.DMA,
]


def _fill_rows(buf, value):
    @pl.loop(0, K)
    def _(r):
        for c in range(0, D, LANES):
            buf[r, pl.ds(c, LANES)] = jnp.full((LANES,), value, jnp.float32)


def _zero_acc(zbuf, acc_sh, sid):
    rbase = sid * RPA
    @pl.loop(0, RPA // K)
    def _(j):
        pltpu.sync_copy(zbuf, acc_sh.at[pl.ds(rbase + j * K, K)])
    pltpu.sync_copy(zbuf.at[pl.ds(0, RPA % K)],
                    acc_sh.at[pl.ds(rbase + (RPA // K) * K, RPA % K)])


def _copyout(acc_sh, out_hbm, cid, sid):
    pltpu.sync_copy(acc_sh.at[pl.ds(sid * RPC, RPC)],
                    out_hbm.at[pl.ds(cid * HALF + sid * RPC, RPC)])


@functools.partial(
    pl.kernel,
    out_type=jax.ShapeDtypeStruct((NPAD, D), jnp.float32),
    mesh=_MESH,
    scratch_types=_SC_SCRATCH,
)
def _sc_agg(g_hbm, row_hbm, col_hbm, out_hbm, ridx, cidx, buf0, buf1,
            zbuf, acc_sh, sem0, sem1):
    cid = lax.axis_index("c")
    sid = lax.axis_index("s")

    def gather(c, buf, sem):
        return pltpu.make_async_copy(g_hbm.at[ridx.at[c]], buf, sem)

    pltpu.sync_copy(row_hbm.at[sid], ridx)
    pltpu.sync_copy(col_hbm.at[cid].at[sid], cidx)
    _fill_rows(zbuf, 0.0)
    _zero_acc(zbuf, acc_sh, sid)
    plsc.subcore_barrier()

    @pl.loop(0, NCH)
    def _(i):
        pltpu.async_copy(g_hbm.at[ridx.at[i]], buf0, sem0).wait()
        pltpu.sync_copy(buf0, acc_sh.at[cidx.at[i]], add=True)

    plsc.subcore_barrier()
    _copyout(acc_sh, out_hbm, cid, sid)


@functools.partial(
    pl.kernel,
    out_type=jax.ShapeDtypeStruct((NPAD, D), jnp.float32),
    mesh=_MESH,
    scratch_types=_SC_SCRATCH,
)
def _sc_deg(row_hbm, col_hbm, out_hbm, ridx, cidx, buf, buf1, zbuf,
            acc_sh, sem0, sem1):
    cid = lax.axis_index("c")
    sid = lax.axis_index("s")

    pltpu.sync_copy(col_hbm.at[cid].at[sid], cidx)
    _fill_rows(buf, 1.0)
    _fill_rows(zbuf, 0.0)
    _zero_acc(zbuf, acc_sh, sid)
    plsc.subcore_barrier()

    @pl.loop(0, NCH)
    def _(i):
        pltpu.sync_copy(buf, acc_sh.at[cidx.at[i]], add=True)

    plsc.subcore_barrier()
    _copyout(acc_sh, out_hbm, cid, sid)


BM = 1000
EROWS = E // D


def _clamp_body(row_ref, col_ref, ro_ref, co_ref):
    row = row_ref[...]
    col = col_ref[...]
    junk = HALF + (col & (NJ - 1))
    for c in range(NC):
        lcl = col - c * HALF
        ok = (lcl >= 0) & (lcl < HALF)
        ro_ref[c] = row
        co_ref[c] = jnp.where(ok, lcl, junk)


def _tc_clamp(row2d, col2d):
    return pl.pallas_call(
        _clamp_body,
        grid=(1,),
        in_specs=[pl.BlockSpec((EROWS, D), lambda i: (0, 0)),
                  pl.BlockSpec((EROWS, D), lambda i: (0, 0))],
        out_specs=[pl.BlockSpec((NC, EROWS, D), lambda i: (0, 0, 0)),
                   pl.BlockSpec((NC, EROWS, D), lambda i: (0, 0, 0))],
        out_shape=[jax.ShapeDtypeStruct((NC, EROWS, D), jnp.int32),
                   jax.ShapeDtypeStruct((NC, EROWS, D), jnp.int32)],
    )(row2d, col2d)


def _mm_body(x_ref, w_ref, o_ref):
    o_ref[...] = jnp.dot(x_ref[...], w_ref[...],
                         preferred_element_type=jnp.float32)


def _tc_mm(x, W):
    return pl.pallas_call(
        _mm_body,
        grid=(N // BM,),
        in_specs=[pl.BlockSpec((BM, D), lambda i: (i, 0)),
                  pl.BlockSpec((D, D), lambda i: (0, 0))],
        out_specs=pl.BlockSpec((BM, D), lambda i: (i, 0)),
        out_shape=jax.ShapeDtypeStruct((N, D), jnp.float32),
    )(x, W)


def _prep_body(degc_ref, h_ref, dinv_ref, g_ref):
    deg = degc_ref[:, 0:1] + 1.0
    dinv = lax.rsqrt(deg)
    dinv_ref[...] = dinv
    g_ref[...] = dinv * h_ref[...]


def _tc_prep(degc, H1):
    return pl.pallas_call(
        _prep_body,
        grid=(N // BM,),
        in_specs=[pl.BlockSpec((BM, D), lambda i: (i, 0)),
                  pl.BlockSpec((BM, D), lambda i: (i, 0))],
        out_specs=[pl.BlockSpec((BM, 1), lambda i: (i, 0)),
                   pl.BlockSpec((BM, D), lambda i: (i, 0))],
        out_shape=[jax.ShapeDtypeStruct((N, 1), jnp.float32),
                   jax.ShapeDtypeStruct((N, D), jnp.float32)],
    )(degc, H1)


def _mid_mm_body(s_ref, g_ref, dinv_ref, b_ref, w_ref, o_ref):
    dinv = dinv_ref[...]
    t = jnp.maximum(dinv * (s_ref[...] + g_ref[...]) + b_ref[...], 0.0)
    o_ref[...] = dinv * jnp.dot(t, w_ref[...],
                                preferred_element_type=jnp.float32)


def _mid_nomm_body(s_ref, g_ref, dinv_ref, b_ref, o_ref):
    dinv = dinv_ref[...]
    t = jnp.maximum(dinv * (s_ref[...] + g_ref[...]) + b_ref[...], 0.0)
    o_ref[...] = dinv * t


def _tc_mid(S, g, dinv, b, W=None):
    specs = [pl.BlockSpec((BM, D), lambda i: (i, 0)),
             pl.BlockSpec((BM, D), lambda i: (i, 0)),
             pl.BlockSpec((BM, 1), lambda i: (i, 0)),
             pl.BlockSpec((1, D), lambda i: (0, 0))]
    args = [S, g, dinv, b.reshape(1, D)]
    if W is not None:
        specs.append(pl.BlockSpec((D, D), lambda i: (0, 0)))
        args.append(W)
    return pl.pallas_call(
        _mid_mm_body if W is not None else _mid_nomm_body,
        grid=(N // BM,),
        in_specs=specs,
        out_specs=pl.BlockSpec((BM, D), lambda i: (i, 0)),
        out_shape=jax.ShapeDtypeStruct((N, D), jnp.float32),
    )(*args)


def _fin_body(s_ref, g_ref, dinv_ref, w_ref, b_ref, o_ref):
    pre = dinv_ref[...] * (s_ref[...] + g_ref[...])
    o_ref[...] = (jnp.dot(pre, w_ref[...], preferred_element_type=jnp.float32)
                  + b_ref[...])


def _tc_fin(S3, g3, dinv, W3, b3):
    return pl.pallas_call(
        _fin_body,
        grid=(N // BM,),
        in_specs=[pl.BlockSpec((BM, D), lambda i: (i, 0)),
                  pl.BlockSpec((BM, D), lambda i: (i, 0)),
                  pl.BlockSpec((BM, 1), lambda i: (i, 0)),
                  pl.BlockSpec((D, DOUT), lambda i: (0, 0)),
                  pl.BlockSpec((1, DOUT), lambda i: (0, 0))],
        out_specs=pl.BlockSpec((BM, DOUT), lambda i: (i, 0)),
        out_shape=jax.ShapeDtypeStruct((N, DOUT), jnp.float32),
    )(S3, g3, dinv, W3, b3.reshape(1, DOUT))


def kernel(x, edge_index, W1, b1, W2, b2, W3, b3):
    row = edge_index[0].astype(jnp.int32)
    col = edge_index[1].astype(jnp.int32)
    rowc, colc = _tc_clamp(row.reshape(EROWS, D), col.reshape(EROWS, D))
    rowt = row.reshape(NS, NCH, K)
    colc = colc.reshape(NC, NS, NCH, K)

    degf = _sc_deg(rowt, colc)
    H1 = _tc_mm(x, W1)
    dinv, g1 = _tc_prep(degf[:N], H1)
    S1 = _sc_agg(g1, rowt, colc)
    g2 = _tc_mid(S1[:N], g1, dinv, b1, W2)
    S2 = _sc_agg(g2, rowt, colc)
    g3 = _tc_mid(S2[:N], g2, dinv, b2)
    S3 = _sc_agg(g3, rowt, colc)
    return _tc_fin(S3[:N], g3, dinv, W3, b3)


# speedup vs baseline: 30.4008x; 1.1501x over previous
import functools

import jax
import jax.numpy as jnp
from jax import lax
from jax.experimental import pallas as pl
from jax.experimental.pallas import tpu as pltpu
from jax.experimental.pallas import tpu_sc as plsc

N = 10000
E = 320000
D = 128
DOUT = 40
NC, NS, LANES = 2, 16, 16
HALF = 5120
NJ = 128
NACC = HALF + NJ
EPT = E // NS
K = 128
EPTP = 20096
NCH = EPTP // K
RPA = NACC // NS
RPC = HALF // NS
NPAD = NC * HALF

_MESH = plsc.VectorSubcoreMesh(core_axis_name="c", subcore_axis_name="s")

_SC_SCRATCH = [
    pltpu.VMEM((NCH, K), jnp.int32),
    pltpu.VMEM((NCH, K), jnp.int32),
    pltpu.VMEM((K, D), jnp.float32),
    pltpu.VMEM((K, D), jnp.float32),
    pltpu.VMEM((K, D), jnp.float32),
    pltpu.VMEM_SHARED((NACC, D), jnp.float32),
    pltpu.SemaphoreType.DMA,
    pltpu.SemaphoreType.DMA,
]


def _fill_rows(buf, value):
    @pl.loop(0, K)
    def _(r):
        for c in range(0, D, LANES):
            buf[r, pl.ds(c, LANES)] = jnp.full((LANES,), value, jnp.float32)


def _zero_acc(zbuf, acc_sh, sid):
    rbase = sid * RPA
    @pl.loop(0, RPA // K)
    def _(j):
        pltpu.sync_copy(zbuf, acc_sh.at[pl.ds(rbase + j * K, K)])
    pltpu.sync_copy(zbuf.at[pl.ds(0, RPA % K)],
                    acc_sh.at[pl.ds(rbase + (RPA // K) * K, RPA % K)])


def _pad_idx(a, pad_vals):
    pad = jnp.tile(pad_vals[None, :], (NS, 1)).astype(jnp.int32)
    return jnp.concatenate([a, pad], axis=1).reshape(NS, NCH, K)


def _copyout(acc_sh, out_hbm, cid, sid):
    pltpu.sync_copy(acc_sh.at[pl.ds(sid * RPC, RPC)],
                    out_hbm.at[pl.ds(cid * HALF + sid * RPC, RPC)])


@functools.partial(
    pl.kernel,
    out_type=jax.ShapeDtypeStruct((NPAD, D), jnp.float32),
    mesh=_MESH,
    scratch_types=_SC_SCRATCH,
)
def _sc_agg(g_hbm, row_hbm, col_hbm, out_hbm, ridx, cidx, buf0, buf1,
            zbuf, acc_sh, sem0, sem1):
    cid = lax.axis_index("c")
    sid = lax.axis_index("s")

    def gather(c, buf, sem):
        return pltpu.make_async_copy(g_hbm.at[ridx.at[c]], buf, sem)

    pltpu.sync_copy(row_hbm.at[sid], ridx)
    pltpu.sync_copy(col_hbm.at[cid].at[sid], cidx)
    _fill_rows(zbuf, 0.0)
    _zero_acc(zbuf, acc_sh, sid)
    plsc.subcore_barrier()

    @pl.loop(0, NCH)
    def _(i):
        pltpu.async_copy(g_hbm.at[ridx.at[i]], buf0, sem0).wait()
        pltpu.sync_copy(buf0, acc_sh.at[cidx.at[i]], add=True)

    plsc.subcore_barrier()
    _copyout(acc_sh, out_hbm, cid, sid)


@functools.partial(
    pl.kernel,
    out_type=jax.ShapeDtypeStruct((NPAD, D), jnp.float32),
    mesh=_MESH,
    scratch_types=_SC_SCRATCH,
)
def _sc_deg(row_hbm, col_hbm, out_hbm, ridx, cidx, buf, buf1, zbuf,
            acc_sh, sem0, sem1):
    cid = lax.axis_index("c")
    sid = lax.axis_index("s")

    pltpu.sync_copy(col_hbm.at[cid].at[sid], cidx)
    _fill_rows(buf, 1.0)
    _fill_rows(zbuf, 0.0)
    _zero_acc(zbuf, acc_sh, sid)
    plsc.subcore_barrier()

    @pl.loop(0, NCH)
    def _(i):
        pltpu.sync_copy(buf, acc_sh.at[cidx.at[i]], add=True)

    plsc.subcore_barrier()
    _copyout(acc_sh, out_hbm, cid, sid)


BM = 1000
EROWS = E // D


def _clamp_body(row_ref, col_ref, ro_ref, co_ref):
    row = row_ref[...]
    col = col_ref[...]
    junk = HALF + (col & (NJ - 1))
    for c in range(NC):
        lcl = col - c * HALF
        ok = (lcl >= 0) & (lcl < HALF)
        ro_ref[c] = row
        co_ref[c] = jnp.where(ok, lcl, junk)


def _tc_clamp(row2d, col2d):
    return pl.pallas_call(
        _clamp_body,
        grid=(1,),
        in_specs=[pl.BlockSpec((EROWS, D), lambda i: (0, 0)),
                  pl.BlockSpec((EROWS, D), lambda i: (0, 0))],
        out_specs=[pl.BlockSpec((NC, EROWS, D), lambda i: (0, 0, 0)),
                   pl.BlockSpec((NC, EROWS, D), lambda i: (0, 0, 0))],
        out_shape=[jax.ShapeDtypeStruct((NC, EROWS, D), jnp.int32),
                   jax.ShapeDtypeStruct((NC, EROWS, D), jnp.int32)],
    )(row2d, col2d)


def _mm_body(x_ref, w_ref, o_ref):
    o_ref[...] = jnp.dot(x_ref[...], w_ref[...],
                         preferred_element_type=jnp.float32)


def _tc_mm(x, W):
    return pl.pallas_call(
        _mm_body,
        grid=(N // BM,),
        in_specs=[pl.BlockSpec((BM, D), lambda i: (i, 0)),
                  pl.BlockSpec((D, D), lambda i: (0, 0))],
        out_specs=pl.BlockSpec((BM, D), lambda i: (i, 0)),
        out_shape=jax.ShapeDtypeStruct((N, D), jnp.float32),
    )(x, W)


def _prep_body(degc_ref, h_ref, dinv_ref, g_ref):
    deg = degc_ref[:, 0:1] + 1.0
    dinv = lax.rsqrt(deg)
    dinv_ref[...] = dinv
    g_ref[...] = dinv * h_ref[...]


def _tc_prep(degc, H1):
    return pl.pallas_call(
        _prep_body,
        grid=(N // BM,),
        in_specs=[pl.BlockSpec((BM, D), lambda i: (i, 0)),
                  pl.BlockSpec((BM, D), lambda i: (i, 0))],
        out_specs=[pl.BlockSpec((BM, 1), lambda i: (i, 0)),
                   pl.BlockSpec((BM, D), lambda i: (i, 0))],
        out_shape=[jax.ShapeDtypeStruct((N, 1), jnp.float32),
                   jax.ShapeDtypeStruct((N, D), jnp.float32)],
    )(degc, H1)


def _mid_mm_body(s_ref, g_ref, dinv_ref, b_ref, w_ref, o_ref):
    dinv = dinv_ref[...]
    t = jnp.maximum(dinv * (s_ref[...] + g_ref[...]) + b_ref[...], 0.0)
    o_ref[...] = dinv * jnp.dot(t, w_ref[...],
                                preferred_element_type=jnp.float32)


def _mid_nomm_body(s_ref, g_ref, dinv_ref, b_ref, o_ref):
    dinv = dinv_ref[...]
    t = jnp.maximum(dinv * (s_ref[...] + g_ref[...]) + b_ref[...], 0.0)
    o_ref[...] = dinv * t


def _tc_mid(S, g, dinv, b, W=None):
    specs = [pl.BlockSpec((BM, D), lambda i: (i, 0)),
             pl.BlockSpec((BM, D), lambda i: (i, 0)),
             pl.BlockSpec((BM, 1), lambda i: (i, 0)),
             pl.BlockSpec((1, D), lambda i: (0, 0))]
    args = [S, g, dinv, b.reshape(1, D)]
    if W is not None:
        specs.append(pl.BlockSpec((D, D), lambda i: (0, 0)))
        args.append(W)
    return pl.pallas_call(
        _mid_mm_body if W is not None else _mid_nomm_body,
        grid=(N // BM,),
        in_specs=specs,
        out_specs=pl.BlockSpec((BM, D), lambda i: (i, 0)),
        out_shape=jax.ShapeDtypeStruct((N, D), jnp.float32),
    )(*args)


def _fin_body(s_ref, g_ref, dinv_ref, w_ref, b_ref, o_ref):
    pre = dinv_ref[...] * (s_ref[...] + g_ref[...])
    o_ref[...] = (jnp.dot(pre, w_ref[...], preferred_element_type=jnp.float32)
                  + b_ref[...])


def _tc_fin(S3, g3, dinv, W3, b3):
    return pl.pallas_call(
        _fin_body,
        grid=(N // BM,),
        in_specs=[pl.BlockSpec((BM, D), lambda i: (i, 0)),
                  pl.BlockSpec((BM, D), lambda i: (i, 0)),
                  pl.BlockSpec((BM, 1), lambda i: (i, 0)),
                  pl.BlockSpec((D, DOUT), lambda i: (0, 0)),
                  pl.BlockSpec((1, DOUT), lambda i: (0, 0))],
        out_specs=pl.BlockSpec((BM, DOUT), lambda i: (i, 0)),
        out_shape=jax.ShapeDtypeStruct((N, DOUT), jnp.float32),
    )(S3, g3, dinv, W3, b3.reshape(1, DOUT))


def kernel(x, edge_index, W1, b1, W2, b2, W3, b3):
    row = edge_index[0].astype(jnp.int32)
    col = edge_index[1].astype(jnp.int32)
    rowc, colc = _tc_clamp(row.reshape(EROWS, D), col.reshape(EROWS, D))
    npad_e = EPTP - EPT
    rowt = _pad_idx(row.reshape(NS, EPT), jnp.arange(npad_e))
    junkpad = HALF + (jnp.arange(npad_e) % NJ)
    colc = jnp.stack([_pad_idx(colc[c].reshape(NS, EPT), junkpad)
                      for c in range(NC)])

    degf = _sc_deg(rowt, colc)
    H1 = _tc_mm(x, W1)
    dinv, g1 = _tc_prep(degf[:N], H1)
    S1 = _sc_agg(g1, rowt, colc)
    g2 = _tc_mid(S1[:N], g1, dinv, b1, W2)
    S2 = _sc_agg(g2, rowt, colc)
    g3 = _tc_mid(S2[:N], g2, dinv, b2)
    S3 = _sc_agg(g3, rowt, colc)
    return _tc_fin(S3[:N], g3, dinv, W3, b3)
